# baseline (speedup 1.0000x reference)
import functools

import jax
import jax.numpy as jnp
from jax import lax
from jax.experimental import pallas as pl
from jax.experimental.pallas import tpu as pltpu
from jax.experimental.pallas import tpu_sc as plsc

_INPUT_DIM = 39
_N_CAT = 26
_VOCAB = 100000
_EMB = 16
_BATCH = 16384
_N_CONT = _INPUT_DIM - _N_CAT
_OUT_DIM = _N_CONT + _N_CAT * _EMB

_NC = 2
_NS = 16
_NW = _NC * _NS

_B_PER_W = _BATCH // _NW
_CHUNK = 128
_N_CHUNKS = _B_PER_W // _CHUNK

_L = 16

_NJ_A = _N_CAT // 2
_NJ_B = _N_CAT - _NJ_A
_ROWS_A = _N_CONT + _NJ_A * _EMB


def _make_embed(j0, nj, pass_through_rows):
    ncont = _N_CONT if pass_through_rows == 0 else 0
    row0 = pass_through_rows + ncont
    n_rows = row0 + nj * _EMB

    @functools.partial(
        pl.kernel,
        mesh=plsc.VectorSubcoreMesh(core_axis_name="c", subcore_axis_name="s"),
        out_type=jax.ShapeDtypeStruct((n_rows, _BATCH), jnp.float32),
        scratch_types=[
            pltpu.VMEM((_CHUNK, _INPUT_DIM), jnp.int32),
            pltpu.VMEM((nj * _CHUNK,), jnp.int32),
            pltpu.VMEM((n_rows, _CHUNK), jnp.float32),
            pltpu.SemaphoreType.DMA,
        ],
        compiler_params=pltpu.CompilerParams(
            use_tc_tiling_on_sc=False, needs_layout_passes=False
        ),
    )
    def _embed(x_hbm, tabt_hbm, *rest):
        if pass_through_rows:
            prev_hbm, out_hbm, x_v, idx_v, out_v, sem = rest
        else:
            prev_hbm = None
            out_hbm, x_v, idx_v, out_v, sem = rest
        wid = lax.axis_index("s") * _NC + lax.axis_index("c")
        w0 = wid * _B_PER_W
        iota = lax.iota(jnp.int32, _L)

        def chunk_body(c, carry):
            b0 = w0 + c * _CHUNK
            pltpu.sync_copy(x_hbm.at[pl.ds(b0, _CHUNK)], x_v)

            if pass_through_rows:
                prev_cp = pltpu.async_copy(
                    prev_hbm.at[:, pl.ds(b0, _CHUNK)],
                    out_v.at[pl.ds(0, pass_through_rows)],
                    sem,
                )

            def feat_body(j, carry2):
                for g in range(_CHUNK // _L):
                    rb = g * _L + iota
                    r = plsc.load_gather(
                        x_v, [rb, iota * 0 + (_N_CONT + j0 + j)])
                    idx_v[pl.ds(j * _CHUNK + g * _L, _L)] = r
                for e in range(_EMB):
                    pltpu.async_copy(
                        tabt_hbm.at[j, e].at[
                            idx_v.at[pl.ds(j * _CHUNK, _CHUNK)]],
                        out_v.at[row0 + j * _EMB + e],
                        sem,
                    )
                return carry2

            lax.fori_loop(0, nj, feat_body, 0)

            for col in range(ncont):
                for g in range(_CHUNK // _L):
                    rb = g * _L + iota
                    vals = plsc.load_gather(x_v, [rb, iota * 0 + col])
                    out_v[col, pl.ds(g * _L, _L)] = vals.astype(jnp.float32)

            def drain_body(k, carry2):
                pltpu.make_async_copy(
                    tabt_hbm.at[0, 0, pl.ds(0, _CHUNK)],
                    out_v.at[row0],
                    sem,
                ).wait()
                return carry2

            lax.fori_loop(0, nj * _EMB, drain_body, 0)
            if pass_through_rows:
                prev_cp.wait()

            pltpu.sync_copy(out_v, out_hbm.at[:, pl.ds(b0, _CHUNK)])
            return carry

        lax.fori_loop(0, _N_CHUNKS, chunk_body, 0)

    return _embed


_embed_a = _make_embed(0, _NJ_A, 0)
_embed_b = _make_embed(_NJ_A, _NJ_B, _ROWS_A)


def kernel(x, tables):
    out_a = _embed_a(x, tables[:_NJ_A].transpose(0, 2, 1))
    out_t = _embed_b(x, tables[_NJ_A:].transpose(0, 2, 1), out_a)
    return out_t.T


# speedup vs baseline: 1.0626x; 1.0626x over previous
import functools

import jax
import jax.numpy as jnp
from jax import lax
from jax.experimental import pallas as pl
from jax.experimental.pallas import tpu as pltpu
from jax.experimental.pallas import tpu_sc as plsc

_INPUT_DIM = 39
_N_CAT = 26
_VOCAB = 100000
_EMB = 16
_BATCH = 16384
_N_CONT = _INPUT_DIM - _N_CAT
_OUT_DIM = _N_CONT + _N_CAT * _EMB

_NC = 2
_NS = 16
_NW = _NC * _NS

_B_PER_W = _BATCH // _NW
_CHUNK = 128
_N_CHUNKS = _B_PER_W // _CHUNK

_L = 16


@functools.partial(
    pl.kernel,
    mesh=plsc.VectorSubcoreMesh(core_axis_name="c", subcore_axis_name="s"),
    out_type=jax.ShapeDtypeStruct((_OUT_DIM, _BATCH), jnp.float32),
    scratch_types=[
        pltpu.VMEM((_CHUNK, _INPUT_DIM), jnp.int32),
        pltpu.VMEM((_N_CAT * _CHUNK,), jnp.int32),
        pltpu.VMEM((_OUT_DIM, _CHUNK), jnp.float32),
        pltpu.SemaphoreType.DMA,
    ],
    compiler_params=pltpu.CompilerParams(
        use_tc_tiling_on_sc=False, needs_layout_passes=False
    ),
)
def _sc_embed(x_hbm, tabt_hbm, out_hbm, x_v, idx_v, out_v, sem):
    wid = lax.axis_index("s") * _NC + lax.axis_index("c")
    w0 = wid * _B_PER_W
    iota = lax.iota(jnp.int32, _L)

    def chunk_body(c, carry):
        b0 = w0 + c * _CHUNK
        pltpu.sync_copy(x_hbm.at[pl.ds(b0, _CHUNK)], x_v)

        def feat_body(j, carry2):
            for g in range(_CHUNK // _L):
                rb = g * _L + iota
                r = plsc.load_gather(x_v, [rb, iota * 0 + (_N_CONT + j)])
                idx_v[pl.ds(j * _CHUNK + g * _L, _L)] = r
            for e in range(_EMB):
                pltpu.async_copy(
                    tabt_hbm.at[j, e].at[idx_v.at[pl.ds(j * _CHUNK, _CHUNK)]],
                    out_v.at[_N_CONT + j * _EMB + e],
                    sem,
                )
            return carry2

        lax.fori_loop(0, _N_CAT, feat_body, 0)

        for col in range(_N_CONT):
            for g in range(_CHUNK // _L):
                rb = g * _L + iota
                vals = plsc.load_gather(x_v, [rb, iota * 0 + col])
                out_v[col, pl.ds(g * _L, _L)] = vals.astype(jnp.float32)

        def drain_body(k, carry2):
            pltpu.make_async_copy(
                tabt_hbm.at[0, 0, pl.ds(0, _CHUNK)],
                out_v.at[_N_CONT],
                sem,
            ).wait()
            return carry2

        lax.fori_loop(0, _N_CAT * _EMB, drain_body, 0)

        pltpu.sync_copy(out_v, out_hbm.at[:, pl.ds(b0, _CHUNK)])
        return carry

    lax.fori_loop(0, _N_CHUNKS, chunk_body, 0)


def kernel(x, tables):
    out_t = _sc_embed(x, tables.transpose(0, 2, 1))
    return out_t.T


# speedup vs baseline: 1.0994x; 1.0347x over previous
import functools

import jax
import jax.numpy as jnp
from jax import lax
from jax.experimental import pallas as pl
from jax.experimental.pallas import tpu as pltpu
from jax.experimental.pallas import tpu_sc as plsc

_INPUT_DIM = 39
_N_CAT = 26
_VOCAB = 100000
_EMB = 16
_BATCH = 16384
_N_CONT = _INPUT_DIM - _N_CAT
_OUT_DIM = _N_CONT + _N_CAT * _EMB

_NC = 2
_NS = 16
_NW = _NC * _NS

_B_PER_W = _BATCH // _NW
_CHUNK = 128
_N_CHUNKS = _B_PER_W // _CHUNK

_L = 16


@functools.partial(
    pl.kernel,
    mesh=plsc.VectorSubcoreMesh(core_axis_name="c", subcore_axis_name="s"),
    out_type=jax.ShapeDtypeStruct((_OUT_DIM, _BATCH), jnp.float32),
    scratch_types=[
        pltpu.VMEM((_INPUT_DIM, _CHUNK), jnp.int32),
        pltpu.VMEM((_OUT_DIM, _CHUNK), jnp.float32),
        pltpu.SemaphoreType.DMA,
    ],
    compiler_params=pltpu.CompilerParams(
        use_tc_tiling_on_sc=False, needs_layout_passes=False
    ),
)
def _sc_embed(xt_hbm, tabt_hbm, out_hbm, xt_v, out_v, sem):
    wid = lax.axis_index("s") * _NC + lax.axis_index("c")
    w0 = wid * _B_PER_W

    def chunk_body(c, carry):
        b0 = w0 + c * _CHUNK
        pltpu.sync_copy(xt_hbm.at[:, pl.ds(b0, _CHUNK)], xt_v)

        def feat_body(j, carry2):
            for e in range(_EMB):
                pltpu.async_copy(
                    tabt_hbm.at[j, e].at[xt_v.at[_N_CONT + j]],
                    out_v.at[_N_CONT + j * _EMB + e],
                    sem,
                )
            return carry2

        lax.fori_loop(0, _N_CAT, feat_body, 0)

        for col in range(_N_CONT):
            for g in range(_CHUNK // _L):
                vals = xt_v[col, pl.ds(g * _L, _L)]
                out_v[col, pl.ds(g * _L, _L)] = vals.astype(jnp.float32)

        def drain_body(k, carry2):
            pltpu.make_async_copy(
                tabt_hbm.at[0, 0, pl.ds(0, _CHUNK)],
                out_v.at[_N_CONT],
                sem,
            ).wait()
            return carry2

        lax.fori_loop(0, _N_CAT * _EMB, drain_body, 0)

        pltpu.sync_copy(out_v, out_hbm.at[:, pl.ds(b0, _CHUNK)])
        return carry

    lax.fori_loop(0, _N_CHUNKS, chunk_body, 0)


def kernel(x, tables):
    out_t = _sc_embed(x.T, tables.transpose(0, 2, 1))
    return out_t.T
